# baseline (speedup 1.0000x reference)
import functools

import jax
import jax.numpy as jnp
from jax import lax
from jax.experimental import pallas as pl
from jax.experimental.pallas import tpu as pltpu
from jax.experimental.pallas import tpu_sc as plsc

N = 10000
E = 160000
F = 128

_LOG2 = 0.6931471805599453


_LOG2E = 1.4426950408889634


def _ssp(z):
    e = jnp.exp2(jnp.abs(z) * (-_LOG2E))
    l = jnp.log2(1.0 + e)
    return jnp.maximum(z, 0.0) + (l - 1.0) * _LOG2


_BE = 1600


def _filter_body(dijk_ref, w1_ref, b1_ref, w2_ref, b2_ref, out_ref):
    h = jnp.dot(dijk_ref[...], w1_ref[...], preferred_element_type=jnp.float32)
    h = _ssp(h + b1_ref[...])
    w = jnp.dot(h, w2_ref[...], preferred_element_type=jnp.float32)
    out_ref[...] = _ssp(w + b2_ref[...])


def _filter(dijk, W1, b1, W2, b2):
    ne = dijk.shape[0]
    return pl.pallas_call(
        _filter_body,
        grid=(ne // _BE,),
        in_specs=[
            pl.BlockSpec((_BE, F), lambda i: (i, 0)),
            pl.BlockSpec((F, F), lambda i: (0, 0)),
            pl.BlockSpec((1, F), lambda i: (0, 0)),
            pl.BlockSpec((F, F), lambda i: (0, 0)),
            pl.BlockSpec((1, F), lambda i: (0, 0)),
        ],
        out_specs=pl.BlockSpec((_BE, F), lambda i: (i, 0)),
        out_shape=jax.ShapeDtypeStruct((ne, F), jnp.float32),
    )(dijk, W1, b1.reshape(1, F), W2, b2.reshape(1, F))


_BN = 1000


def _in2fac_body(x_ref, win_ref, f_ref):
    f_ref[...] = jnp.dot(x_ref[...], win_ref[...],
                         preferred_element_type=jnp.float32)


def _in2fac(x, Win):
    return pl.pallas_call(
        _in2fac_body,
        grid=(N // _BN,),
        in_specs=[
            pl.BlockSpec((_BN, F), lambda i: (i, 0)),
            pl.BlockSpec((F, F), lambda i: (0, 0)),
        ],
        out_specs=pl.BlockSpec((_BN, F), lambda i: (i, 0)),
        out_shape=jax.ShapeDtypeStruct((N, F), jnp.float32),
    )(x, Win)


_NC = 2
_NS = 16
_NW = _NC * _NS
_CHUNK = 40
_EPT = E // _NW
_STEPS = _EPT // _CHUNK
_NBUF = 3
_NPAD = 10240
_ROWS_PER_TILE = _NPAD // _NS


def _make_sc_conv_body(n_steps):
  ept = n_steps * _CHUNK

  def _sc_conv_body(f_hbm, w_hbm, idx_hbm, seg_hbm, zeros_hbm, out_hbm,
                    idx_v, seg_v, rows_v, w_v, conv_sh,
                    sem_i, sem_s, sem_g, sem_w, sem_sc):
    cid = lax.axis_index("c")
    sid = lax.axis_index("s")
    wid = cid * _NS + sid
    ebase = wid * ept

    pltpu.sync_copy(zeros_hbm.at[pl.ds(sid * _ROWS_PER_TILE, _ROWS_PER_TILE)],
                    conv_sh.at[pl.ds(sid * _ROWS_PER_TILE, _ROWS_PER_TILE)])
    plsc.subcore_barrier()

    def issue_idx(k, slot):
        pltpu.async_copy(idx_hbm.at[pl.ds(ebase + k * _CHUNK, _CHUNK)],
                         idx_v.at[slot], sem_i.at[slot])
        pltpu.async_copy(seg_hbm.at[pl.ds(ebase + k * _CHUNK, _CHUNK)],
                         seg_v.at[slot], sem_s.at[slot])

    def wait_idx(slot):
        pltpu.make_async_copy(idx_hbm.at[pl.ds(0, _CHUNK)], idx_v.at[slot],
                              sem_i.at[slot]).wait()
        pltpu.make_async_copy(idx_hbm.at[pl.ds(0, _CHUNK)], seg_v.at[slot],
                              sem_s.at[slot]).wait()

    def issue_loads(k, slot):
        pltpu.async_copy(f_hbm.at[idx_v.at[slot]], rows_v.at[slot],
                         sem_g.at[slot])
        pltpu.async_copy(w_hbm.at[pl.ds(ebase + k * _CHUNK, _CHUNK)],
                         w_v.at[slot], sem_w.at[slot])

    def wait_loads(slot):
        pltpu.make_async_copy(w_hbm.at[pl.ds(0, _CHUNK)], rows_v.at[slot],
                              sem_g.at[slot]).wait()
        pltpu.make_async_copy(w_hbm.at[pl.ds(0, _CHUNK)], w_v.at[slot],
                              sem_w.at[slot]).wait()

    def wait_scatter(slot):
        pltpu.make_async_copy(w_hbm.at[pl.ds(0, _CHUNK)], rows_v.at[slot],
                              sem_sc.at[slot]).wait()

    issue_idx(0, 0)
    issue_idx(1, 1)
    issue_idx(2, 2)
    wait_idx(0)
    issue_loads(0, 0)

    def step(k, _):
        slot = lax.rem(k, _NBUF)
        nslot = lax.rem(k + 1, _NBUF)

        @pl.when(k + 1 < n_steps)
        def _():
            wait_idx(nslot)
            issue_loads(k + 1, nslot)

        wait_loads(slot)

        rs = rows_v.at[slot]
        ws = w_v.at[slot]

        @plsc.parallel_loop(0, _CHUNK, step=1, unroll=4)
        def _mul_row(e):
            for c in range(F // 16):
                sl = pl.ds(c * 16, 16)
                rs[e, sl] = rs[e, sl] * ws[e, sl]

        pltpu.async_copy(rows_v.at[slot], conv_sh.at[seg_v.at[slot]],
                         sem_sc.at[slot], add=True)

        @pl.when(k >= 1)
        def _():
            wait_scatter(lax.rem(k - 1, _NBUF))

            @pl.when(k + 2 < n_steps)
            def _():
                issue_idx(k + 2, lax.rem(k + 2, _NBUF))

        return 0

    lax.fori_loop(0, n_steps, step, 0)
    wait_scatter((n_steps - 1) % _NBUF)
    plsc.subcore_barrier()

    off = sid * _ROWS_PER_TILE
    pltpu.sync_copy(conv_sh.at[pl.ds(off, _ROWS_PER_TILE)],
                    out_hbm.at[cid, pl.ds(off, _ROWS_PER_TILE)])

  return _sc_conv_body


def _sc_conv(f, w, idx_j, seg_i, zeros):
    ne = w.shape[0]
    n_steps = ne // (_NW * _CHUNK)
    mesh = plsc.VectorSubcoreMesh(core_axis_name="c", subcore_axis_name="s")
    kern = functools.partial(
        pl.kernel,
        out_type=jax.ShapeDtypeStruct((_NC, _NPAD, F), jnp.float32),
        mesh=mesh,
        scratch_types=[
            pltpu.VMEM((_NBUF, _CHUNK), jnp.int32),
            pltpu.VMEM((_NBUF, _CHUNK), jnp.int32),
            pltpu.VMEM((_NBUF, _CHUNK, F), jnp.float32),
            pltpu.VMEM((_NBUF, _CHUNK, F), jnp.float32),
            pltpu.VMEM_SHARED((_NPAD, F), jnp.float32),
            pltpu.SemaphoreType.DMA((_NBUF,)),
            pltpu.SemaphoreType.DMA((_NBUF,)),
            pltpu.SemaphoreType.DMA((_NBUF,)),
            pltpu.SemaphoreType.DMA((_NBUF,)),
            pltpu.SemaphoreType.DMA((_NBUF,)),
        ],
    )(_make_sc_conv_body(n_steps))
    return kern(f, w, idx_j, seg_i, zeros)


def _make_epilogue_body(n_parts):
    def _epilogue_body(*refs):
        p_refs = refs[:2 * n_parts]
        x_ref, wout_ref, bout_ref, wd_ref, bd_ref, y_ref, v_ref = refs[2 * n_parts:]
        conv = p_refs[0][0]
        for p in p_refs[1:]:
            conv = conv + p[0]
        c = _ssp(jnp.dot(conv, wout_ref[...], preferred_element_type=jnp.float32)
                 + bout_ref[...])
        v = (jnp.dot(c, wd_ref[...], preferred_element_type=jnp.float32)
             + bd_ref[...])
        v_ref[...] = v
        y_ref[...] = x_ref[...] + v
    return _epilogue_body


def _epilogue(parts, x, Wout, bout, Wd, bd):
    nb = N // _BN
    part_specs = []
    part_args = []
    for p in parts:
        part_specs.append(pl.BlockSpec((1, _BN, F), lambda i: (0, i, 0)))
        part_specs.append(pl.BlockSpec((1, _BN, F), lambda i: (1, i, 0)))
        part_args.extend([p, p])
    return pl.pallas_call(
        _make_epilogue_body(len(parts)),
        grid=(nb,),
        in_specs=part_specs + [
            pl.BlockSpec((_BN, F), lambda i: (i, 0)),
            pl.BlockSpec((F, F), lambda i: (0, 0)),
            pl.BlockSpec((1, F), lambda i: (0, 0)),
            pl.BlockSpec((F, F), lambda i: (0, 0)),
            pl.BlockSpec((1, F), lambda i: (0, 0)),
        ],
        out_specs=[
            pl.BlockSpec((_BN, F), lambda i: (i, 0)),
            pl.BlockSpec((_BN, F), lambda i: (i, 0)),
        ],
        out_shape=[
            jax.ShapeDtypeStruct((N, F), jnp.float32),
            jax.ShapeDtypeStruct((N, F), jnp.float32),
        ],
    )(*part_args, x, Wout, bout.reshape(1, F), Wd, bd.reshape(1, F))


_SLICES = (51200, 51200, 57600)


def kernel(x, dijk, idx_j, seg_i, seg_j, seg_i_sum,
           W1, b1, W2, b2, Win, Wout, bout, Wd, bd):
    f = _in2fac(x, Win)
    zeros = jnp.zeros((_NPAD, F), jnp.float32)
    idx_j = idx_j.astype(jnp.int32)
    seg_i = seg_i.astype(jnp.int32)
    parts = []
    off = 0
    for ne in _SLICES:
        w_s = _filter(lax.slice_in_dim(dijk, off, off + ne), W1, b1, W2, b2)
        parts.append(_sc_conv(f, w_s,
                              lax.slice_in_dim(idx_j, off, off + ne),
                              lax.slice_in_dim(seg_i, off, off + ne), zeros))
        off += ne
    y, v = _epilogue(parts, x, Wout, bout, Wd, bd)
    return (y, v)


# speedup vs baseline: 1.1313x; 1.1313x over previous
import functools

import jax
import jax.numpy as jnp
from jax import lax
from jax.experimental import pallas as pl
from jax.experimental.pallas import tpu as pltpu
from jax.experimental.pallas import tpu_sc as plsc

N = 10000
E = 160000
F = 128

_LOG2 = 0.6931471805599453


_LOG2E = 1.4426950408889634


def _ssp(z):
    e = jnp.exp2(jnp.abs(z) * (-_LOG2E))
    l = jnp.log2(1.0 + e)
    return jnp.maximum(z, 0.0) + (l - 1.0) * _LOG2


_BE = 1600


def _filter_body(dijk_ref, w1_ref, b1_ref, w2_ref, b2_ref, out_ref):
    h = jnp.dot(dijk_ref[...], w1_ref[...], preferred_element_type=jnp.float32)
    h = _ssp(h + b1_ref[...])
    w = jnp.dot(h, w2_ref[...], preferred_element_type=jnp.float32)
    out_ref[...] = _ssp(w + b2_ref[...])


def _filter(dijk, W1, b1, W2, b2, eoff, ne):
    boff = eoff // _BE
    return pl.pallas_call(
        _filter_body,
        grid=(ne // _BE,),
        in_specs=[
            pl.BlockSpec((_BE, F), lambda i, boff=boff: (i + boff, 0)),
            pl.BlockSpec((F, F), lambda i: (0, 0)),
            pl.BlockSpec((1, F), lambda i: (0, 0)),
            pl.BlockSpec((F, F), lambda i: (0, 0)),
            pl.BlockSpec((1, F), lambda i: (0, 0)),
        ],
        out_specs=pl.BlockSpec((_BE, F), lambda i: (i, 0)),
        out_shape=jax.ShapeDtypeStruct((ne, F), jnp.float32),
    )(dijk, W1, b1.reshape(1, F), W2, b2.reshape(1, F))


_BN = 1000


def _in2fac_body(x_ref, win_ref, f_ref):
    f_ref[...] = jnp.dot(x_ref[...], win_ref[...],
                         preferred_element_type=jnp.float32)


def _in2fac(x, Win):
    return pl.pallas_call(
        _in2fac_body,
        grid=(N // _BN,),
        in_specs=[
            pl.BlockSpec((_BN, F), lambda i: (i, 0)),
            pl.BlockSpec((F, F), lambda i: (0, 0)),
        ],
        out_specs=pl.BlockSpec((_BN, F), lambda i: (i, 0)),
        out_shape=jax.ShapeDtypeStruct((N, F), jnp.float32),
    )(x, Win)


_NC = 2
_NS = 16
_NW = _NC * _NS
_CHUNK = 40
_EPT = E // _NW
_STEPS = _EPT // _CHUNK
_NBUF = 3
_NPAD = 10240
_ROWS_PER_TILE = _NPAD // _NS


def _make_sc_conv_body(n_steps, eoff):
  ept = n_steps * _CHUNK

  def _sc_conv_body(f_hbm, w_hbm, idx_hbm, seg_hbm, zeros_hbm, out_hbm,
                    idx_v, seg_v, rows_v, w_v, conv_sh,
                    sem_i, sem_s, sem_g, sem_w, sem_sc):
    cid = lax.axis_index("c")
    sid = lax.axis_index("s")
    wid = cid * _NS + sid
    wbase = wid * ept
    ebase = eoff + wbase

    pltpu.sync_copy(zeros_hbm.at[pl.ds(sid * _ROWS_PER_TILE, _ROWS_PER_TILE)],
                    conv_sh.at[pl.ds(sid * _ROWS_PER_TILE, _ROWS_PER_TILE)])
    plsc.subcore_barrier()

    def issue_idx(k, slot):
        pltpu.async_copy(idx_hbm.at[pl.ds(ebase + k * _CHUNK, _CHUNK)],
                         idx_v.at[slot], sem_i.at[slot])
        pltpu.async_copy(seg_hbm.at[pl.ds(ebase + k * _CHUNK, _CHUNK)],
                         seg_v.at[slot], sem_s.at[slot])

    def wait_idx(slot):
        pltpu.make_async_copy(idx_hbm.at[pl.ds(0, _CHUNK)], idx_v.at[slot],
                              sem_i.at[slot]).wait()
        pltpu.make_async_copy(idx_hbm.at[pl.ds(0, _CHUNK)], seg_v.at[slot],
                              sem_s.at[slot]).wait()

    def issue_loads(k, slot):
        pltpu.async_copy(f_hbm.at[idx_v.at[slot]], rows_v.at[slot],
                         sem_g.at[slot])
        pltpu.async_copy(w_hbm.at[pl.ds(wbase + k * _CHUNK, _CHUNK)],
                         w_v.at[slot], sem_w.at[slot])

    def wait_loads(slot):
        pltpu.make_async_copy(w_hbm.at[pl.ds(0, _CHUNK)], rows_v.at[slot],
                              sem_g.at[slot]).wait()
        pltpu.make_async_copy(w_hbm.at[pl.ds(0, _CHUNK)], w_v.at[slot],
                              sem_w.at[slot]).wait()

    def wait_scatter(slot):
        pltpu.make_async_copy(w_hbm.at[pl.ds(0, _CHUNK)], rows_v.at[slot],
                              sem_sc.at[slot]).wait()

    issue_idx(0, 0)
    issue_idx(1, 1)
    issue_idx(2, 2)
    wait_idx(0)
    issue_loads(0, 0)

    def step(k, _):
        slot = lax.rem(k, _NBUF)
        nslot = lax.rem(k + 1, _NBUF)

        @pl.when(k + 1 < n_steps)
        def _():
            wait_idx(nslot)
            issue_loads(k + 1, nslot)

        wait_loads(slot)

        rs = rows_v.at[slot]
        ws = w_v.at[slot]

        @plsc.parallel_loop(0, _CHUNK, step=1, unroll=4)
        def _mul_row(e):
            for c in range(F // 16):
                sl = pl.ds(c * 16, 16)
                rs[e, sl] = rs[e, sl] * ws[e, sl]

        pltpu.async_copy(rows_v.at[slot], conv_sh.at[seg_v.at[slot]],
                         sem_sc.at[slot], add=True)

        @pl.when(k >= 1)
        def _():
            wait_scatter(lax.rem(k - 1, _NBUF))

            @pl.when(k + 2 < n_steps)
            def _():
                issue_idx(k + 2, lax.rem(k + 2, _NBUF))

        return 0

    lax.fori_loop(0, n_steps, step, 0)
    wait_scatter((n_steps - 1) % _NBUF)
    plsc.subcore_barrier()

    off = sid * _ROWS_PER_TILE
    pltpu.sync_copy(conv_sh.at[pl.ds(off, _ROWS_PER_TILE)],
                    out_hbm.at[cid, pl.ds(off, _ROWS_PER_TILE)])

  return _sc_conv_body


def _sc_conv(f, w, idx_j, seg_i, zeros, eoff):
    ne = w.shape[0]
    n_steps = ne // (_NW * _CHUNK)
    mesh = plsc.VectorSubcoreMesh(core_axis_name="c", subcore_axis_name="s")
    kern = functools.partial(
        pl.kernel,
        out_type=jax.ShapeDtypeStruct((_NC, _NPAD, F), jnp.float32),
        mesh=mesh,
        scratch_types=[
            pltpu.VMEM((_NBUF, _CHUNK), jnp.int32),
            pltpu.VMEM((_NBUF, _CHUNK), jnp.int32),
            pltpu.VMEM((_NBUF, _CHUNK, F), jnp.float32),
            pltpu.VMEM((_NBUF, _CHUNK, F), jnp.float32),
            pltpu.VMEM_SHARED((_NPAD, F), jnp.float32),
            pltpu.SemaphoreType.DMA((_NBUF,)),
            pltpu.SemaphoreType.DMA((_NBUF,)),
            pltpu.SemaphoreType.DMA((_NBUF,)),
            pltpu.SemaphoreType.DMA((_NBUF,)),
            pltpu.SemaphoreType.DMA((_NBUF,)),
        ],
    )(_make_sc_conv_body(n_steps, eoff))
    return kern(f, w, idx_j, seg_i, zeros)


def _make_epilogue_body(n_parts):
    def _epilogue_body(*refs):
        p_refs = refs[:2 * n_parts]
        x_ref, wout_ref, bout_ref, wd_ref, bd_ref, y_ref, v_ref = refs[2 * n_parts:]
        conv = p_refs[0][0]
        for p in p_refs[1:]:
            conv = conv + p[0]
        c = _ssp(jnp.dot(conv, wout_ref[...], preferred_element_type=jnp.float32)
                 + bout_ref[...])
        v = (jnp.dot(c, wd_ref[...], preferred_element_type=jnp.float32)
             + bd_ref[...])
        v_ref[...] = v
        y_ref[...] = x_ref[...] + v
    return _epilogue_body


def _epilogue(parts, x, Wout, bout, Wd, bd):
    nb = N // _BN
    part_specs = []
    part_args = []
    for p in parts:
        part_specs.append(pl.BlockSpec((1, _BN, F), lambda i: (0, i, 0)))
        part_specs.append(pl.BlockSpec((1, _BN, F), lambda i: (1, i, 0)))
        part_args.extend([p, p])
    return pl.pallas_call(
        _make_epilogue_body(len(parts)),
        grid=(nb,),
        in_specs=part_specs + [
            pl.BlockSpec((_BN, F), lambda i: (i, 0)),
            pl.BlockSpec((F, F), lambda i: (0, 0)),
            pl.BlockSpec((1, F), lambda i: (0, 0)),
            pl.BlockSpec((F, F), lambda i: (0, 0)),
            pl.BlockSpec((1, F), lambda i: (0, 0)),
        ],
        out_specs=[
            pl.BlockSpec((_BN, F), lambda i: (i, 0)),
            pl.BlockSpec((_BN, F), lambda i: (i, 0)),
        ],
        out_shape=[
            jax.ShapeDtypeStruct((N, F), jnp.float32),
            jax.ShapeDtypeStruct((N, F), jnp.float32),
        ],
    )(*part_args, x, Wout, bout.reshape(1, F), Wd, bd.reshape(1, F))


_SLICES = (38400, 57600, 64000)


def kernel(x, dijk, idx_j, seg_i, seg_j, seg_i_sum,
           W1, b1, W2, b2, Win, Wout, bout, Wd, bd):
    f = _in2fac(x, Win)
    zeros = jnp.zeros((_NPAD, F), jnp.float32)
    idx_j = idx_j.astype(jnp.int32)
    seg_i = seg_i.astype(jnp.int32)
    parts = []
    off = 0
    for ne in _SLICES:
        w_s = _filter(dijk, W1, b1, W2, b2, off, ne)
        parts.append(_sc_conv(f, w_s, idx_j, seg_i, zeros, off))
        off += ne
    y, v = _epilogue(parts, x, Wout, bout, Wd, bd)
    return (y, v)


# speedup vs baseline: 1.2280x; 1.0855x over previous
import functools

import jax
import jax.numpy as jnp
from jax import lax
from jax.experimental import pallas as pl
from jax.experimental.pallas import tpu as pltpu
from jax.experimental.pallas import tpu_sc as plsc

N = 10000
E = 160000
F = 128

_LOG2 = 0.6931471805599453


_LOG2E = 1.4426950408889634


def _ssp(z):
    e = jnp.exp2(jnp.abs(z) * (-_LOG2E))
    l = jnp.log2(1.0 + e)
    return jnp.maximum(z, 0.0) + (l - 1.0) * _LOG2


_BE = 1600


def _filter_body(dijk_ref, w1_ref, b1_ref, w2_ref, b2_ref, out_ref):
    h = jnp.dot(dijk_ref[...], w1_ref[...], preferred_element_type=jnp.float32)
    h = _ssp(h + b1_ref[...])
    w = jnp.dot(h, w2_ref[...], preferred_element_type=jnp.float32)
    out_ref[...] = _ssp(w + b2_ref[...])


def _filter(dijk, W1, b1, W2, b2, eoff, ne):
    boff = eoff // _BE
    return pl.pallas_call(
        _filter_body,
        grid=(ne // _BE,),
        in_specs=[
            pl.BlockSpec((_BE, F), lambda i, boff=boff: (i + boff, 0)),
            pl.BlockSpec((F, F), lambda i: (0, 0)),
            pl.BlockSpec((1, F), lambda i: (0, 0)),
            pl.BlockSpec((F, F), lambda i: (0, 0)),
            pl.BlockSpec((1, F), lambda i: (0, 0)),
        ],
        out_specs=pl.BlockSpec((_BE, F), lambda i: (i, 0)),
        out_shape=jax.ShapeDtypeStruct((ne, F), jnp.float32),
    )(dijk, W1, b1.reshape(1, F), W2, b2.reshape(1, F))


_BN = 1000


def _in2fac_body(x_ref, win_ref, f_ref):
    f_ref[...] = jnp.dot(x_ref[...], win_ref[...],
                         preferred_element_type=jnp.float32)


def _in2fac(x, Win):
    return pl.pallas_call(
        _in2fac_body,
        grid=(N // _BN,),
        in_specs=[
            pl.BlockSpec((_BN, F), lambda i: (i, 0)),
            pl.BlockSpec((F, F), lambda i: (0, 0)),
        ],
        out_specs=pl.BlockSpec((_BN, F), lambda i: (i, 0)),
        out_shape=jax.ShapeDtypeStruct((N, F), jnp.float32),
    )(x, Win)


_NC = 2
_NS = 16
_NW = _NC * _NS
_CHUNK = 40
_EPT = E // _NW
_STEPS = _EPT // _CHUNK
_NBUF = 3
_NPAD = 10240
_ROWS_PER_TILE = _NPAD // _NS


def _make_sc_conv_body(n_steps, eoff):
  ept = n_steps * _CHUNK

  def _sc_conv_body(f_hbm, w_hbm, idx_hbm, seg_hbm, zeros_hbm, out_hbm,
                    idx_v, seg_v, rows_v, w_v, conv_sh,
                    sem_i, sem_s, sem_g, sem_w, sem_sc):
    cid = lax.axis_index("c")
    sid = lax.axis_index("s")
    wid = cid * _NS + sid
    wbase = wid * ept
    ebase = eoff + wbase

    pltpu.sync_copy(zeros_hbm.at[pl.ds(sid * _ROWS_PER_TILE, _ROWS_PER_TILE)],
                    conv_sh.at[pl.ds(sid * _ROWS_PER_TILE, _ROWS_PER_TILE)])
    plsc.subcore_barrier()

    def issue_idx(k, slot):
        pltpu.async_copy(idx_hbm.at[pl.ds(ebase + k * _CHUNK, _CHUNK)],
                         idx_v.at[slot], sem_i.at[slot])
        pltpu.async_copy(seg_hbm.at[pl.ds(ebase + k * _CHUNK, _CHUNK)],
                         seg_v.at[slot], sem_s.at[slot])

    def wait_idx(slot):
        pltpu.make_async_copy(idx_hbm.at[pl.ds(0, _CHUNK)], idx_v.at[slot],
                              sem_i.at[slot]).wait()
        pltpu.make_async_copy(idx_hbm.at[pl.ds(0, _CHUNK)], seg_v.at[slot],
                              sem_s.at[slot]).wait()

    def issue_loads(k, slot):
        pltpu.async_copy(f_hbm.at[idx_v.at[slot]], rows_v.at[slot],
                         sem_g.at[slot])
        pltpu.async_copy(w_hbm.at[pl.ds(wbase + k * _CHUNK, _CHUNK)],
                         w_v.at[slot], sem_w.at[slot])

    def wait_loads(slot):
        pltpu.make_async_copy(w_hbm.at[pl.ds(0, _CHUNK)], rows_v.at[slot],
                              sem_g.at[slot]).wait()
        pltpu.make_async_copy(w_hbm.at[pl.ds(0, _CHUNK)], w_v.at[slot],
                              sem_w.at[slot]).wait()

    def wait_scatter(slot):
        pltpu.make_async_copy(w_hbm.at[pl.ds(0, _CHUNK)], rows_v.at[slot],
                              sem_sc.at[slot]).wait()

    issue_idx(0, 0)
    issue_idx(1, 1)
    issue_idx(2, 2)
    wait_idx(0)
    issue_loads(0, 0)

    def step(k, _):
        slot = lax.rem(k, _NBUF)
        nslot = lax.rem(k + 1, _NBUF)

        @pl.when(k + 1 < n_steps)
        def _():
            wait_idx(nslot)
            issue_loads(k + 1, nslot)

        wait_loads(slot)

        rs = rows_v.at[slot]
        ws = w_v.at[slot]

        @plsc.parallel_loop(0, _CHUNK, step=1, unroll=4)
        def _mul_row(e):
            for c in range(F // 16):
                sl = pl.ds(c * 16, 16)
                rs[e, sl] = rs[e, sl] * ws[e, sl]

        pltpu.async_copy(rows_v.at[slot], conv_sh.at[seg_v.at[slot]],
                         sem_sc.at[slot], add=True)

        @pl.when(k >= 1)
        def _():
            wait_scatter(lax.rem(k - 1, _NBUF))

            @pl.when(k + 2 < n_steps)
            def _():
                issue_idx(k + 2, lax.rem(k + 2, _NBUF))

        return 0

    lax.fori_loop(0, n_steps, step, 0)
    wait_scatter((n_steps - 1) % _NBUF)
    plsc.subcore_barrier()

    off = sid * _ROWS_PER_TILE
    pltpu.sync_copy(conv_sh.at[pl.ds(off, _ROWS_PER_TILE)],
                    out_hbm.at[cid, pl.ds(off, _ROWS_PER_TILE)])

  return _sc_conv_body


def _sc_conv(f, w, idx_j, seg_i, zeros, eoff):
    ne = w.shape[0]
    n_steps = ne // (_NW * _CHUNK)
    mesh = plsc.VectorSubcoreMesh(core_axis_name="c", subcore_axis_name="s")
    kern = functools.partial(
        pl.kernel,
        out_type=jax.ShapeDtypeStruct((_NC, _NPAD, F), jnp.float32),
        mesh=mesh,
        scratch_types=[
            pltpu.VMEM((_NBUF, _CHUNK), jnp.int32),
            pltpu.VMEM((_NBUF, _CHUNK), jnp.int32),
            pltpu.VMEM((_NBUF, _CHUNK, F), jnp.float32),
            pltpu.VMEM((_NBUF, _CHUNK, F), jnp.float32),
            pltpu.VMEM_SHARED((_NPAD, F), jnp.float32),
            pltpu.SemaphoreType.DMA((_NBUF,)),
            pltpu.SemaphoreType.DMA((_NBUF,)),
            pltpu.SemaphoreType.DMA((_NBUF,)),
            pltpu.SemaphoreType.DMA((_NBUF,)),
            pltpu.SemaphoreType.DMA((_NBUF,)),
        ],
    )(_make_sc_conv_body(n_steps, eoff))
    return kern(f, w, idx_j, seg_i, zeros)


def _make_epilogue_body(n_parts):
    def _epilogue_body(*refs):
        p_refs = refs[:2 * n_parts]
        x_ref, wout_ref, bout_ref, wd_ref, bd_ref, y_ref, v_ref = refs[2 * n_parts:]
        conv = p_refs[0][0]
        for p in p_refs[1:]:
            conv = conv + p[0]
        c = _ssp(jnp.dot(conv, wout_ref[...], preferred_element_type=jnp.float32)
                 + bout_ref[...])
        v = (jnp.dot(c, wd_ref[...], preferred_element_type=jnp.float32)
             + bd_ref[...])
        v_ref[...] = v
        y_ref[...] = x_ref[...] + v
    return _epilogue_body


def _epilogue(parts, x, Wout, bout, Wd, bd):
    nb = N // _BN
    part_specs = []
    part_args = []
    for p in parts:
        part_specs.append(pl.BlockSpec((1, _BN, F), lambda i: (0, i, 0)))
        part_specs.append(pl.BlockSpec((1, _BN, F), lambda i: (1, i, 0)))
        part_args.extend([p, p])
    return pl.pallas_call(
        _make_epilogue_body(len(parts)),
        grid=(nb,),
        in_specs=part_specs + [
            pl.BlockSpec((_BN, F), lambda i: (i, 0)),
            pl.BlockSpec((F, F), lambda i: (0, 0)),
            pl.BlockSpec((1, F), lambda i: (0, 0)),
            pl.BlockSpec((F, F), lambda i: (0, 0)),
            pl.BlockSpec((1, F), lambda i: (0, 0)),
        ],
        out_specs=[
            pl.BlockSpec((_BN, F), lambda i: (i, 0)),
            pl.BlockSpec((_BN, F), lambda i: (i, 0)),
        ],
        out_shape=[
            jax.ShapeDtypeStruct((N, F), jnp.float32),
            jax.ShapeDtypeStruct((N, F), jnp.float32),
        ],
    )(*part_args, x, Wout, bout.reshape(1, F), Wd, bd.reshape(1, F))


_SLICES = (32000, 51200, 76800)


def kernel(x, dijk, idx_j, seg_i, seg_j, seg_i_sum,
           W1, b1, W2, b2, Win, Wout, bout, Wd, bd):
    f = _in2fac(x, Win)
    zeros = jnp.zeros((_NPAD, F), jnp.float32)
    idx_j = idx_j.astype(jnp.int32)
    seg_i = seg_i.astype(jnp.int32)
    parts = []
    off = 0
    dijk_in = dijk
    for ne in _SLICES:
        w_s = _filter(dijk_in, W1, b1, W2, b2, off, ne)
        parts.append(_sc_conv(f, w_s, idx_j, seg_i, zeros, off))
        dijk_in, _ = lax.optimization_barrier((dijk, w_s))
        off += ne
    y, v = _epilogue(parts, x, Wout, bout, Wd, bd)
    return (y, v)
